# baseline (speedup 1.0000x reference)
import functools

import jax
import jax.numpy as jnp
from jax import lax
from jax.experimental import pallas as pl
from jax.experimental.pallas import tpu as pltpu
from jax.experimental.pallas import tpu_sc as plsc

N = 50000
D = 128
E = 400000

NW = 32
NPT = 1568
NPAD = NW * NPT
NSUB = 4
SUBN = NPT // NSUB
EC = 4000
NCHUNK = E // EC
BLK = 128
LCAP = 14464
SCAP = 3968

_NODE_BLK = 1000
_NEG_INF = float("-inf")


def _mm_body(x_ref, w1_ref, b1_ref, w2_ref, b2_ref,
             a1_ref, a2_ref, bo1_ref, bo2_ref):
    xb = x_ref[...]
    for w_ref, b_ref, a_out, b_out in (
            (w1_ref, b1_ref, a1_ref, bo1_ref),
            (w2_ref, b2_ref, a2_ref, bo2_ref)):
        w = w_ref[...]
        wa = w[:, :D]
        wb = w[:, D:]
        a_out[...] = jax.lax.dot_general(
            xb, wa - wb, (((1,), (1,)), ((), ())),
            preferred_element_type=jnp.float32,
            precision=jax.lax.Precision.HIGHEST) + b_ref[...]
        b_out[...] = jax.lax.dot_general(
            xb, wb, (((1,), (1,)), ((), ())),
            preferred_element_type=jnp.float32,
            precision=jax.lax.Precision.HIGHEST)


def _tc_matmuls(x, w1, b1, w2, b2):
    nblk = N // _NODE_BLK
    full = pl.BlockSpec((_NODE_BLK, D), lambda i: (i, 0))
    wspec = pl.BlockSpec((D, 2 * D), lambda i: (0, 0))
    bspec = pl.BlockSpec((D,), lambda i: (0,))
    return pl.pallas_call(
        _mm_body,
        grid=(nblk,),
        in_specs=[full, wspec, bspec, wspec, bspec],
        out_specs=[full, full, full, full],
        out_shape=[jax.ShapeDtypeStruct((N, D), jnp.float32)] * 4,
    )(x, w1, b1, w2, b2)


def _combine_body(a1_ref, m1_ref, a2_ref, m2_ref, o_ref):
    o_ref[...] = (jnp.maximum(a1_ref[...] + m1_ref[...], 0.0)
                  + jnp.maximum(a2_ref[...] + m2_ref[...], 0.0))


def _tc_combine(a1, m1, a2, m2):
    nblk = N // _NODE_BLK
    spec = pl.BlockSpec((_NODE_BLK, D), lambda i: (i, 0))
    return pl.pallas_call(
        _combine_body,
        grid=(nblk,),
        in_specs=[spec, spec, spec, spec],
        out_specs=spec,
        out_shape=jax.ShapeDtypeStruct((N, D), jnp.float32),
    )(a1, m1, a2, m2)


def _sc_body(src1_hbm, dst1_hbm, src2_hbm, dst2_hbm, b1_hbm, b2_hbm,
             m1_hbm, m2_hbm,
             ebuf, lmain, lsub, gidx, gbuf, acc, claim):
    wid = lax.axis_index("s") * 2 + lax.axis_index("c")
    lo = wid * NPT
    iota = lax.iota(jnp.int32, 16)
    neg_inf16 = jnp.full((16,), _NEG_INF, jnp.float32)
    zeros16 = jnp.zeros((16,), jnp.int32)

    for tp in range(2):
        src_hbm = (src1_hbm, src2_hbm)[tp]
        dst_hbm = (dst1_hbm, dst2_hbm)[tp]
        b_hbm = (b1_hbm, b2_hbm)[tp]
        m_hbm = (m1_hbm, m2_hbm)[tp]

        def filt_chunk(c, cnt_v):
            pltpu.sync_copy(dst_hbm.at[pl.ds(c * EC, EC)],
                            ebuf.at[pl.ds(0, EC)])
            pltpu.sync_copy(src_hbm.at[pl.ds(c * EC, EC)],
                            ebuf.at[pl.ds(EC, EC)])

            def filt_vec(k, cnt):
                d = ebuf[pl.ds(k * 16, 16)]
                s = ebuf[pl.ds(EC + k * 16, 16)]
                ld = d - lo
                m = (ld >= 0) & (ld < NPT)
                mi = m.astype(jnp.int32)
                pos = cnt + plsc.cumsum(mi) - mi
                m = m & (pos < LCAP)
                plsc.store_scatter(lmain, [pos],
                                   s | (ld << 16), mask=m)
                return cnt + plsc.all_reduce_population_count(m)

            return lax.fori_loop(0, EC // 16, filt_vec, cnt_v)

        cnt_v = lax.fori_loop(0, NCHUNK, filt_chunk, zeros16)
        count = lax.reduce_max_p.bind(cnt_v, axes=(0,))

        def split_vec(k, cnts):
            p = lmain[pl.ds(k * 16, 16)]
            valid = iota < (count - k * 16)
            ld = p >> 16
            new_cnts = []
            for s_i in range(NSUB):
                ms = valid & (ld >= s_i * SUBN) & (ld < (s_i + 1) * SUBN)
                mi = ms.astype(jnp.int32)
                pos = cnts[s_i] + plsc.cumsum(mi) - mi
                ms = ms & (pos < SCAP)
                plsc.store_scatter(lsub, [pos + s_i * (SCAP + BLK)],
                                   p - ((s_i * SUBN) << 16), mask=ms)
                new_cnts.append(cnts[s_i]
                                + plsc.all_reduce_population_count(ms))
            return tuple(new_cnts)

        cnts = lax.fori_loop(0, (count + 15) // 16, split_vec,
                             (zeros16,) * NSUB)

        for s_i in range(NSUB):
            subcnt = lax.reduce_max_p.bind(cnts[s_i], axes=(0,))
            sbase = s_i * (SCAP + BLK)
            nblocks = (subcnt + BLK - 1) // BLK

            def init_vec(r, _):
                for c in range(D // 16):
                    acc[r, pl.ds(c * 16, 16)] = neg_inf16
                return 0
            lax.fori_loop(0, SUBN, init_vec, 0)

            def do_block(b, _b):
                def unpack_vec(k, _k):
                    p = lsub[pl.ds(sbase + b * BLK + k * 16, 16)]
                    gidx[pl.ds(k * 16, 16)] = jnp.minimum(p & 0xFFFF, N - 1)
                    return 0
                lax.fori_loop(0, BLK // 16, unpack_vec, 0, unroll=True)
                pltpu.sync_copy(b_hbm.at[gidx], gbuf)

                def do_group(g, _g):
                    base = b * BLK + g * 16
                    p = lsub[pl.ds(sbase + base, 16)]
                    ld = p >> 16
                    valid = iota < (subcnt - base)
                    plsc.store_scatter(claim, [ld], iota, mask=valid)
                    won = plsc.load_gather(claim, [ld],
                                           mask=valid) == iota
                    batch = valid & won
                    qv = iota + g * 16

                    def do_f(f, fv):
                        gval = plsc.load_gather(gbuf, [qv, fv], mask=batch)
                        aval = plsc.load_gather(acc, [ld, fv], mask=batch)
                        plsc.store_scatter(acc, [ld, fv],
                                           jnp.maximum(aval, gval),
                                           mask=batch)
                        return fv + 1
                    lax.fori_loop(0, D, do_f, zeros16, unroll=16)

                    remdup = valid & ~won
                    ndup = lax.reduce_max_p.bind(
                        plsc.all_reduce_population_count(remdup), axes=(0,))

                    def do_dup(i, rd):
                        lane = plsc.all_reduce_ffs(rd)
                        mthis = iota == lane

                        def do_fd(f, fv):
                            gval = plsc.load_gather(gbuf, [qv, fv],
                                                    mask=mthis)
                            aval = plsc.load_gather(acc, [ld, fv],
                                                    mask=mthis)
                            plsc.store_scatter(acc, [ld, fv],
                                               jnp.maximum(aval, gval),
                                               mask=mthis)
                            return fv + 1
                        lax.fori_loop(0, D, do_fd, zeros16, unroll=16)
                        return rd & ~mthis
                    lax.fori_loop(0, ndup, do_dup, remdup)
                    return 0
                lax.fori_loop(0, BLK // 16, do_group, 0)
                return 0
            lax.fori_loop(0, nblocks, do_block, 0)

            pltpu.sync_copy(acc,
                            m_hbm.at[pl.ds(lo + s_i * SUBN, SUBN)])


@functools.partial(
    pl.kernel,
    compiler_params=pltpu.CompilerParams(needs_layout_passes=False),
    out_type=(jax.ShapeDtypeStruct((NPAD, D), jnp.float32),
              jax.ShapeDtypeStruct((NPAD, D), jnp.float32)),
    mesh=plsc.VectorSubcoreMesh(core_axis_name="c", subcore_axis_name="s",
                                num_cores=2, num_subcores=16),
    scratch_types=[
        pltpu.VMEM((2 * EC,), jnp.int32),
        pltpu.VMEM((LCAP + 16,), jnp.int32),
        pltpu.VMEM((NSUB * (SCAP + BLK),), jnp.int32),
        pltpu.VMEM((BLK,), jnp.int32),
        pltpu.VMEM((BLK, D), jnp.float32),
        pltpu.VMEM((SUBN, D), jnp.float32),
        pltpu.VMEM((SUBN,), jnp.int32),
    ],
)
def _sc_segmax(*args):
    _sc_body(*args)


def kernel(x, edge_index_tp, edge_index_int, W1, b1, W2, b2):
    a1, a2, bm1, bm2 = _tc_matmuls(x, W1, b1, W2, b2)
    m1p, m2p = _sc_segmax(edge_index_tp[0], edge_index_tp[1],
                          edge_index_int[0], edge_index_int[1], bm1, bm2)
    return _tc_combine(a1, m1p[:N], a2, m2p[:N])


# speedup vs baseline: 2.7971x; 2.7971x over previous
import functools

import jax
import jax.numpy as jnp
from jax import lax
from jax.experimental import pallas as pl
from jax.experimental.pallas import tpu as pltpu
from jax.experimental.pallas import tpu_sc as plsc

N = 50000
D = 128
E = 400000

NW = 32
NPT = 1568
NPAD = NW * NPT
NSUB = 4
SUBN = NPT // NSUB
EC = 4000
NCHUNK = E // EC
BLK = 128
LCAP = 14464
SCAP = 3968

_NODE_BLK = 1000
_NEG_INF = float("-inf")


def _mm_body(x_ref, w1_ref, b1_ref, w2_ref, b2_ref,
             a1_ref, a2_ref, bo1_ref, bo2_ref):
    xb = x_ref[...]
    for w_ref, b_ref, a_out, b_out in (
            (w1_ref, b1_ref, a1_ref, bo1_ref),
            (w2_ref, b2_ref, a2_ref, bo2_ref)):
        w = w_ref[...]
        wa = w[:, :D]
        wb = w[:, D:]
        a_out[...] = jax.lax.dot_general(
            xb, wa - wb, (((1,), (1,)), ((), ())),
            preferred_element_type=jnp.float32,
            precision=jax.lax.Precision.HIGHEST) + b_ref[...]
        b_out[...] = jax.lax.dot_general(
            xb, wb, (((1,), (1,)), ((), ())),
            preferred_element_type=jnp.float32,
            precision=jax.lax.Precision.HIGHEST)


def _tc_matmuls(x, w1, b1, w2, b2):
    nblk = N // _NODE_BLK
    full = pl.BlockSpec((_NODE_BLK, D), lambda i: (i, 0))
    wspec = pl.BlockSpec((D, 2 * D), lambda i: (0, 0))
    bspec = pl.BlockSpec((D,), lambda i: (0,))
    return pl.pallas_call(
        _mm_body,
        grid=(nblk,),
        in_specs=[full, wspec, bspec, wspec, bspec],
        out_specs=[full, full, full, full],
        out_shape=[jax.ShapeDtypeStruct((N, D), jnp.float32)] * 4,
    )(x, w1, b1, w2, b2)


def _combine_body(a1_ref, m1_ref, a2_ref, m2_ref, o_ref):
    o_ref[...] = (jnp.maximum(a1_ref[...] + m1_ref[...], 0.0)
                  + jnp.maximum(a2_ref[...] + m2_ref[...], 0.0))


def _tc_combine(a1, m1, a2, m2):
    nblk = N // _NODE_BLK
    spec = pl.BlockSpec((_NODE_BLK, D), lambda i: (i, 0))
    return pl.pallas_call(
        _combine_body,
        grid=(nblk,),
        in_specs=[spec, spec, spec, spec],
        out_specs=spec,
        out_shape=jax.ShapeDtypeStruct((N, D), jnp.float32),
    )(a1, m1, a2, m2)


def _sc_body(src1_hbm, dst1_hbm, src2_hbm, dst2_hbm, b1_hbm, b2_hbm,
             m1_hbm, m2_hbm,
             ebuf, lmain, lsub, gidx, gbuf, acc):
    wid = lax.axis_index("s") * 2 + lax.axis_index("c")
    lo = wid * NPT
    iota = lax.iota(jnp.int32, 16)
    neg_inf16 = jnp.full((16,), _NEG_INF, jnp.float32)
    zeros16 = jnp.zeros((16,), jnp.int32)

    for tp in range(2):
        src_hbm = (src1_hbm, src2_hbm)[tp]
        dst_hbm = (dst1_hbm, dst2_hbm)[tp]
        b_hbm = (b1_hbm, b2_hbm)[tp]
        m_hbm = (m1_hbm, m2_hbm)[tp]

        def filt_chunk(c, cnt_v):
            pltpu.sync_copy(dst_hbm.at[pl.ds(c * EC, EC)],
                            ebuf.at[pl.ds(0, EC)])
            pltpu.sync_copy(src_hbm.at[pl.ds(c * EC, EC)],
                            ebuf.at[pl.ds(EC, EC)])

            def filt_vec(k, cnt):
                d = ebuf[pl.ds(k * 16, 16)]
                s = ebuf[pl.ds(EC + k * 16, 16)]
                ld = d - lo
                m = (ld >= 0) & (ld < NPT)
                mi = m.astype(jnp.int32)
                pos = cnt + plsc.cumsum(mi) - mi
                m = m & (pos < LCAP)
                plsc.store_scatter(lmain, [pos],
                                   s | (ld << 16), mask=m)
                return cnt + plsc.all_reduce_population_count(m)

            return lax.fori_loop(0, EC // 16, filt_vec, cnt_v)

        cnt_v = lax.fori_loop(0, NCHUNK, filt_chunk, zeros16)
        count = lax.reduce_max_p.bind(cnt_v, axes=(0,))

        def split_vec(k, cnts):
            p = lmain[pl.ds(k * 16, 16)]
            valid = iota < (count - k * 16)
            ld = p >> 16
            new_cnts = []
            for s_i in range(NSUB):
                ms = valid & (ld >= s_i * SUBN) & (ld < (s_i + 1) * SUBN)
                mi = ms.astype(jnp.int32)
                pos = cnts[s_i] + plsc.cumsum(mi) - mi
                ms = ms & (pos < SCAP)
                plsc.store_scatter(lsub, [pos + s_i * (SCAP + BLK)],
                                   p - ((s_i * SUBN) << 16), mask=ms)
                new_cnts.append(cnts[s_i]
                                + plsc.all_reduce_population_count(ms))
            return tuple(new_cnts)

        cnts = lax.fori_loop(0, (count + 15) // 16, split_vec,
                             (zeros16,) * NSUB)

        for s_i in range(NSUB):
            subcnt = lax.reduce_max_p.bind(cnts[s_i], axes=(0,))
            sbase = s_i * (SCAP + BLK)
            padv = jnp.full((16,), SUBN << 16, jnp.int32)
            for k in range(BLK // 16):
                plsc.store_scatter(lsub, [cnts[s_i] + sbase + iota + k * 16],
                                   padv)
            nblocks = (subcnt + BLK - 1) // BLK

            def init_vec(r, _):
                for c in range(D // 16):
                    acc[r, pl.ds(c * 16, 16)] = neg_inf16
                return 0
            lax.fori_loop(0, SUBN + 1, init_vec, 0)

            def do_block(b, _b):
                def unpack_vec(k, _k):
                    p = lsub[pl.ds(sbase + b * BLK + k * 16, 16)]
                    gidx[pl.ds(k * 16, 16)] = jnp.minimum(p & 0xFFFF, N - 1)
                    return 0
                lax.fori_loop(0, BLK // 16, unpack_vec, 0, unroll=True)
                pltpu.sync_copy(b_hbm.at[gidx], gbuf)

                def do_group(g, _g):
                    p = lsub[pl.ds(sbase + b * BLK + g * 16, 16)]
                    ld = lax.shift_right_logical(p, 16)
                    for l in range(16):
                        row = lax.reduce_max_p.bind(
                            jnp.where(iota == l, ld, 0), axes=(0,))
                        q = g * 16 + l
                        for c in range(D // 16):
                            gv = gbuf[q, pl.ds(c * 16, 16)]
                            av = acc[row, pl.ds(c * 16, 16)]
                            acc[row, pl.ds(c * 16, 16)] = jnp.maximum(av, gv)
                    return 0
                lax.fori_loop(0, BLK // 16, do_group, 0)
                return 0
            lax.fori_loop(0, nblocks, do_block, 0)

            pltpu.sync_copy(acc.at[pl.ds(0, SUBN)],
                            m_hbm.at[pl.ds(lo + s_i * SUBN, SUBN)])


@functools.partial(
    pl.kernel,
    compiler_params=pltpu.CompilerParams(needs_layout_passes=False),
    out_type=(jax.ShapeDtypeStruct((NPAD, D), jnp.float32),
              jax.ShapeDtypeStruct((NPAD, D), jnp.float32)),
    mesh=plsc.VectorSubcoreMesh(core_axis_name="c", subcore_axis_name="s",
                                num_cores=2, num_subcores=16),
    scratch_types=[
        pltpu.VMEM((2 * EC,), jnp.int32),
        pltpu.VMEM((LCAP + 16,), jnp.int32),
        pltpu.VMEM((NSUB * (SCAP + BLK),), jnp.int32),
        pltpu.VMEM((BLK,), jnp.int32),
        pltpu.VMEM((BLK, D), jnp.float32),
        pltpu.VMEM((SUBN + 1, D), jnp.float32),
    ],
)
def _sc_segmax(*args):
    _sc_body(*args)


def kernel(x, edge_index_tp, edge_index_int, W1, b1, W2, b2):
    a1, a2, bm1, bm2 = _tc_matmuls(x, W1, b1, W2, b2)
    m1p, m2p = _sc_segmax(edge_index_tp[0], edge_index_tp[1],
                          edge_index_int[0], edge_index_int[1], bm1, bm2)
    return _tc_combine(a1, m1p[:N], a2, m2p[:N])


# speedup vs baseline: 3.5739x; 1.2777x over previous
import functools

import jax
import jax.numpy as jnp
from jax import lax
from jax.experimental import pallas as pl
from jax.experimental.pallas import tpu as pltpu
from jax.experimental.pallas import tpu_sc as plsc

N = 50000
D = 128
E = 400000

NW = 32
NPT = 1568
NPAD = NW * NPT
NSUB = 4
SUBN = NPT // NSUB
EC = 2000
NCHUNK = E // EC
BLK = 128
LCAP = 14464
SCAP = 3968

_NODE_BLK = 1000
_NEG_INF = float("-inf")


def _mm_body(x_ref, w1_ref, b1_ref, w2_ref, b2_ref,
             a1_ref, a2_ref, bo1_ref, bo2_ref):
    xb = x_ref[...]
    for w_ref, b_ref, a_out, b_out in (
            (w1_ref, b1_ref, a1_ref, bo1_ref),
            (w2_ref, b2_ref, a2_ref, bo2_ref)):
        w = w_ref[...]
        wa = w[:, :D]
        wb = w[:, D:]
        a_out[...] = jax.lax.dot_general(
            xb, wa - wb, (((1,), (1,)), ((), ())),
            preferred_element_type=jnp.float32,
            precision=jax.lax.Precision.HIGHEST) + b_ref[...]
        b_out[...] = jax.lax.dot_general(
            xb, wb, (((1,), (1,)), ((), ())),
            preferred_element_type=jnp.float32,
            precision=jax.lax.Precision.HIGHEST)


def _tc_matmuls(x, w1, b1, w2, b2):
    nblk = N // _NODE_BLK
    full = pl.BlockSpec((_NODE_BLK, D), lambda i: (i, 0))
    wspec = pl.BlockSpec((D, 2 * D), lambda i: (0, 0))
    bspec = pl.BlockSpec((D,), lambda i: (0,))
    return pl.pallas_call(
        _mm_body,
        grid=(nblk,),
        in_specs=[full, wspec, bspec, wspec, bspec],
        out_specs=[full, full, full, full],
        out_shape=[jax.ShapeDtypeStruct((N, D), jnp.float32)] * 4,
    )(x, w1, b1, w2, b2)


def _combine_body(a1_ref, m1_ref, a2_ref, m2_ref, o_ref):
    o_ref[...] = (jnp.maximum(a1_ref[...] + m1_ref[...], 0.0)
                  + jnp.maximum(a2_ref[...] + m2_ref[...], 0.0))


def _tc_combine(a1, m1, a2, m2):
    nblk = N // _NODE_BLK
    spec = pl.BlockSpec((_NODE_BLK, D), lambda i: (i, 0))
    return pl.pallas_call(
        _combine_body,
        grid=(nblk,),
        in_specs=[spec, spec, spec, spec],
        out_specs=spec,
        out_shape=jax.ShapeDtypeStruct((N, D), jnp.float32),
    )(a1, m1, a2, m2)


def _sc_body(src1_hbm, dst1_hbm, src2_hbm, dst2_hbm, b1_hbm, b2_hbm,
             m1_hbm, m2_hbm,
             dbuf0, dbuf1, sbuf0, sbuf1, lmain, lsub,
             gidx0, gidx1, gbuf0, gbuf1, acc, fsem0, fsem1, gsem0, gsem1):
    wid = lax.axis_index("s") * 2 + lax.axis_index("c")
    lo = wid * NPT
    iota = lax.iota(jnp.int32, 16)
    neg_inf16 = jnp.full((16,), _NEG_INF, jnp.float32)
    zeros16 = jnp.zeros((16,), jnp.int32)

    for tp in range(2):
        src_hbm = (src1_hbm, src2_hbm)[tp]
        dst_hbm = (dst1_hbm, dst2_hbm)[tp]
        b_hbm = (b1_hbm, b2_hbm)[tp]
        m_hbm = (m1_hbm, m2_hbm)[tp]

        def chunk_start(c, db, sb, sem):
            pltpu.make_async_copy(dst_hbm.at[pl.ds(c * EC, EC)], db,
                                  sem).start()
            pltpu.make_async_copy(src_hbm.at[pl.ds(c * EC, EC)], sb,
                                  sem).start()

        def chunk_wait(c, db, sb, sem):
            pltpu.make_async_copy(dst_hbm.at[pl.ds(c * EC, EC)], db,
                                  sem).wait()
            pltpu.make_async_copy(src_hbm.at[pl.ds(c * EC, EC)], sb,
                                  sem).wait()

        def scan_chunk(db, sb, cnt_v):
            def filt_vec(k, cnt):
                d = db[pl.ds(k * 16, 16)]
                s = sb[pl.ds(k * 16, 16)]
                ld = d - lo
                m = plsc.bitcast(ld, jnp.uint32) < jnp.uint32(NPT)
                mi = m.astype(jnp.int32)
                pos = cnt + plsc.cumsum(mi) - mi
                m = m & (pos < LCAP)
                plsc.store_scatter(lmain, [pos],
                                   s | (ld << 16), mask=m)
                return cnt + plsc.all_reduce_population_count(m)
            return lax.fori_loop(0, EC // 16, filt_vec, cnt_v, unroll=4)

        def filt_chunk(c, cnt_v):
            even = (c % 2) == 0

            def do_slot(db, sb, sem, odb, osb, osem):
                chunk_wait(c, db, sb, sem)

                @pl.when(c + 1 < NCHUNK)
                def _():
                    chunk_start(c + 1, odb, osb, osem)
                return scan_chunk(db, sb, cnt_v)

            r0 = lax.cond(even,
                          lambda: do_slot(dbuf0, sbuf0, fsem0,
                                          dbuf1, sbuf1, fsem1),
                          lambda: do_slot(dbuf1, sbuf1, fsem1,
                                          dbuf0, sbuf0, fsem0))
            return r0

        chunk_start(0, dbuf0, sbuf0, fsem0)
        cnt_v = lax.fori_loop(0, NCHUNK, filt_chunk, zeros16)
        count = lax.reduce_max_p.bind(cnt_v, axes=(0,))

        def split_vec(k, cnts):
            p = lmain[pl.ds(k * 16, 16)]
            valid = iota < (count - k * 16)
            ld = p >> 16
            new_cnts = []
            for s_i in range(NSUB):
                ms = valid & (ld >= s_i * SUBN) & (ld < (s_i + 1) * SUBN)
                mi = ms.astype(jnp.int32)
                pos = cnts[s_i] + plsc.cumsum(mi) - mi
                ms = ms & (pos < SCAP)
                plsc.store_scatter(lsub, [pos + s_i * (SCAP + BLK)],
                                   p - ((s_i * SUBN) << 16), mask=ms)
                new_cnts.append(cnts[s_i]
                                + plsc.all_reduce_population_count(ms))
            return tuple(new_cnts)

        cnts = lax.fori_loop(0, (count + 15) // 16, split_vec,
                             (zeros16,) * NSUB)

        padv = jnp.full((16,), SUBN << 16, jnp.int32)
        for s_i in range(NSUB):
            for k in range(BLK // 16):
                plsc.store_scatter(
                    lsub, [cnts[s_i] + s_i * (SCAP + BLK) + iota + k * 16],
                    padv)
        scv = zeros16
        for s_i in range(NSUB):
            scv = jnp.where(iota == s_i, cnts[s_i], scv)

        def sub_range(s_i, _s):
            subcnt = lax.reduce_max_p.bind(
                jnp.where(iota == s_i, scv, 0), axes=(0,))
            sbase = s_i * (SCAP + BLK)
            nblocks = (subcnt + BLK - 1) // BLK

            def init_vec(r, _):
                for c in range(D // 16):
                    acc[r, pl.ds(c * 16, 16)] = neg_inf16
                return 0
            lax.fori_loop(0, SUBN + 1, init_vec, 0)

            def unpack(b, gi):
                def unpack_vec(k, _k):
                    p = lsub[pl.ds(sbase + b * BLK + k * 16, 16)]
                    gi[pl.ds(k * 16, 16)] = jnp.minimum(p & 0xFFFF, N - 1)
                    return 0
                lax.fori_loop(0, BLK // 16, unpack_vec, 0, unroll=True)

            def compute(b, gb):
                def do_group(g, _g):
                    p = lsub[pl.ds(sbase + b * BLK + g * 16, 16)]
                    ld = lax.shift_right_logical(p, 16)
                    rows = [lax.reduce_max_p.bind(
                        jnp.where(iota == l, ld, 0), axes=(0,))
                        for l in range(16)]
                    for l in range(16):
                        q = g * 16 + l
                        row = rows[l]
                        for c in range(D // 16):
                            gv = gb[q, pl.ds(c * 16, 16)]
                            av = acc[row, pl.ds(c * 16, 16)]
                            acc[row, pl.ds(c * 16, 16)] = jnp.maximum(av, gv)
                    return 0
                lax.fori_loop(0, BLK // 16, do_group, 0)

            def do_block(b, _b):
                even = (b % 2) == 0

                def slot(gi, gb, sem, ogi, ogb, osem):
                    pltpu.make_async_copy(b_hbm.at[gi], gb, sem).wait()

                    @pl.when(b + 1 < nblocks)
                    def _():
                        unpack(b + 1, ogi)
                        pltpu.make_async_copy(b_hbm.at[ogi], ogb,
                                              osem).start()
                    compute(b, gb)
                    return 0

                return lax.cond(even,
                                lambda: slot(gidx0, gbuf0, gsem0,
                                             gidx1, gbuf1, gsem1),
                                lambda: slot(gidx1, gbuf1, gsem1,
                                             gidx0, gbuf0, gsem0))

            @pl.when(nblocks > 0)
            def _():
                unpack(0, gidx0)
                pltpu.make_async_copy(b_hbm.at[gidx0], gbuf0, gsem0).start()

            lax.fori_loop(0, nblocks, do_block, 0)

            pltpu.sync_copy(acc.at[pl.ds(0, SUBN)],
                            m_hbm.at[pl.ds(lo + s_i * SUBN, SUBN)])
            return 0

        lax.fori_loop(0, NSUB, sub_range, 0)


@functools.partial(
    pl.kernel,
    compiler_params=pltpu.CompilerParams(needs_layout_passes=False),
    out_type=(jax.ShapeDtypeStruct((NPAD, D), jnp.float32),
              jax.ShapeDtypeStruct((NPAD, D), jnp.float32)),
    mesh=plsc.VectorSubcoreMesh(core_axis_name="c", subcore_axis_name="s",
                                num_cores=2, num_subcores=16),
    scratch_types=[
        pltpu.VMEM((EC,), jnp.int32),
        pltpu.VMEM((EC,), jnp.int32),
        pltpu.VMEM((EC,), jnp.int32),
        pltpu.VMEM((EC,), jnp.int32),
        pltpu.VMEM((LCAP + 16,), jnp.int32),
        pltpu.VMEM((NSUB * (SCAP + BLK),), jnp.int32),
        pltpu.VMEM((BLK,), jnp.int32),
        pltpu.VMEM((BLK,), jnp.int32),
        pltpu.VMEM((BLK, D), jnp.float32),
        pltpu.VMEM((BLK, D), jnp.float32),
        pltpu.VMEM((SUBN + 1, D), jnp.float32),
        pltpu.SemaphoreType.DMA,
        pltpu.SemaphoreType.DMA,
        pltpu.SemaphoreType.DMA,
        pltpu.SemaphoreType.DMA,
    ],
)
def _sc_segmax(*args):
    _sc_body(*args)


def kernel(x, edge_index_tp, edge_index_int, W1, b1, W2, b2):
    a1, a2, bm1, bm2 = _tc_matmuls(x, W1, b1, W2, b2)
    m1p, m2p = _sc_segmax(edge_index_tp[0], edge_index_tp[1],
                          edge_index_int[0], edge_index_int[1], bm1, bm2)
    return _tc_combine(a1, m1p[:N], a2, m2p[:N])


# speedup vs baseline: 4.1429x; 1.1592x over previous
import functools

import jax
import jax.numpy as jnp
from jax import lax
from jax.experimental import pallas as pl
from jax.experimental.pallas import tpu as pltpu
from jax.experimental.pallas import tpu_sc as plsc

N = 50000
D = 128
E = 400000

NW = 32
NPT = 1568
NPAD = NW * NPT
NSUB = 4
SUBN = NPT // NSUB
EC = 2000
NCHUNK = E // EC
BLK = 128
LSUB = 1088
SCAP = 3968

_NODE_BLK = 1000
_NEG_INF = float("-inf")


def _mm_body(x_ref, w1_ref, b1_ref, w2_ref, b2_ref,
             a1_ref, a2_ref, bo1_ref, bo2_ref):
    xb = x_ref[...]
    for w_ref, b_ref, a_out, b_out in (
            (w1_ref, b1_ref, a1_ref, bo1_ref),
            (w2_ref, b2_ref, a2_ref, bo2_ref)):
        w = w_ref[...]
        wa = w[:, :D]
        wb = w[:, D:]
        a_out[...] = jax.lax.dot_general(
            xb, wa - wb, (((1,), (1,)), ((), ())),
            preferred_element_type=jnp.float32,
            precision=jax.lax.Precision.HIGHEST) + b_ref[...]
        b_out[...] = jax.lax.dot_general(
            xb, wb, (((1,), (1,)), ((), ())),
            preferred_element_type=jnp.float32,
            precision=jax.lax.Precision.HIGHEST)


def _tc_matmuls(x, w1, b1, w2, b2):
    nblk = N // _NODE_BLK
    full = pl.BlockSpec((_NODE_BLK, D), lambda i: (i, 0))
    wspec = pl.BlockSpec((D, 2 * D), lambda i: (0, 0))
    bspec = pl.BlockSpec((D,), lambda i: (0,))
    return pl.pallas_call(
        _mm_body,
        grid=(nblk,),
        in_specs=[full, wspec, bspec, wspec, bspec],
        out_specs=[full, full, full, full],
        out_shape=[jax.ShapeDtypeStruct((N, D), jnp.float32)] * 4,
    )(x, w1, b1, w2, b2)


def _combine_body(a1_ref, m1_ref, a2_ref, m2_ref, o_ref):
    o_ref[...] = (jnp.maximum(a1_ref[...] + m1_ref[...], 0.0)
                  + jnp.maximum(a2_ref[...] + m2_ref[...], 0.0))


def _tc_combine(a1, m1, a2, m2):
    nblk = N // _NODE_BLK
    spec = pl.BlockSpec((_NODE_BLK, D), lambda i: (i, 0))
    return pl.pallas_call(
        _combine_body,
        grid=(nblk,),
        in_specs=[spec, spec, spec, spec],
        out_specs=spec,
        out_shape=jax.ShapeDtypeStruct((N, D), jnp.float32),
    )(a1, m1, a2, m2)


def _sc_body(src1_hbm, dst1_hbm, src2_hbm, dst2_hbm, b1_hbm, b2_hbm,
             m1_hbm, m2_hbm,
             dbuf0, dbuf1, sbuf0, sbuf1, lmain, lsub,
             gidx0, gidx1, gbuf0, gbuf1, acc, fsem0, fsem1, gsem0, gsem1):
    wid = lax.axis_index("s") * 2 + lax.axis_index("c")
    lo = wid * NPT
    iota = lax.iota(jnp.int32, 16)
    neg_inf16 = jnp.full((16,), _NEG_INF, jnp.float32)
    zeros16 = jnp.zeros((16,), jnp.int32)

    for tp in range(2):
        src_hbm = (src1_hbm, src2_hbm)[tp]
        dst_hbm = (dst1_hbm, dst2_hbm)[tp]
        b_hbm = (b1_hbm, b2_hbm)[tp]
        m_hbm = (m1_hbm, m2_hbm)[tp]

        def chunk_start(c, db, sb, sem):
            pltpu.make_async_copy(dst_hbm.at[pl.ds(c * EC, EC)], db,
                                  sem).start()
            pltpu.make_async_copy(src_hbm.at[pl.ds(c * EC, EC)], sb,
                                  sem).start()

        def chunk_wait(c, db, sb, sem):
            pltpu.make_async_copy(dst_hbm.at[pl.ds(c * EC, EC)], db,
                                  sem).wait()
            pltpu.make_async_copy(src_hbm.at[pl.ds(c * EC, EC)], sb,
                                  sem).wait()

        lanebase = iota * LSUB

        def scan_chunk(db, sb, cnt_v):
            def filt_vec(k, cnt):
                d = db[pl.ds(k * 16, 16)]
                s = sb[pl.ds(k * 16, 16)]
                ld = d - lo
                m = plsc.bitcast(ld, jnp.uint32) < jnp.uint32(NPT)
                m = m & (cnt < LSUB)
                plsc.store_scatter(lmain, [lanebase + cnt],
                                   s | (ld << 16), mask=m)
                return cnt + m.astype(jnp.int32)
            return lax.fori_loop(0, EC // 16, filt_vec, cnt_v, unroll=4)

        def filt_chunk(c, cnt_v):
            even = (c % 2) == 0

            def do_slot(db, sb, sem, odb, osb, osem):
                chunk_wait(c, db, sb, sem)

                @pl.when(c + 1 < NCHUNK)
                def _():
                    chunk_start(c + 1, odb, osb, osem)
                return scan_chunk(db, sb, cnt_v)

            r0 = lax.cond(even,
                          lambda: do_slot(dbuf0, sbuf0, fsem0,
                                          dbuf1, sbuf1, fsem1),
                          lambda: do_slot(dbuf1, sbuf1, fsem1,
                                          dbuf0, sbuf0, fsem0))
            return r0

        chunk_start(0, dbuf0, sbuf0, fsem0)
        cnt_v = lax.fori_loop(0, NCHUNK, filt_chunk, zeros16)

        def split_lane(l, cnts):
            bound = lax.reduce_max_p.bind(
                jnp.where(iota == l, cnt_v, 0), axes=(0,))

            def split_vec(k, cs):
                p = lmain[pl.ds(l * LSUB + k * 16, 16)]
                valid = iota < (bound - k * 16)
                ld = p >> 16
                new_cs = []
                for s_i in range(NSUB):
                    ms = valid & (ld >= s_i * SUBN) & (ld < (s_i + 1) * SUBN)
                    mi = ms.astype(jnp.int32)
                    pos = cs[s_i] + plsc.cumsum(mi) - mi
                    ms = ms & (pos < SCAP)
                    plsc.store_scatter(lsub, [pos + s_i * (SCAP + BLK)],
                                       p - ((s_i * SUBN) << 16), mask=ms)
                    new_cs.append(cs[s_i]
                                  + plsc.all_reduce_population_count(ms))
                return tuple(new_cs)

            return lax.fori_loop(0, (bound + 15) // 16, split_vec, cnts)

        cnts = lax.fori_loop(0, 16, split_lane, (zeros16,) * NSUB)

        padv = jnp.full((16,), SUBN << 16, jnp.int32)
        for s_i in range(NSUB):
            for k in range(BLK // 16):
                plsc.store_scatter(
                    lsub, [cnts[s_i] + s_i * (SCAP + BLK) + iota + k * 16],
                    padv)
        scv = zeros16
        for s_i in range(NSUB):
            scv = jnp.where(iota == s_i, cnts[s_i], scv)

        def sub_range(s_i, _s):
            subcnt = lax.reduce_max_p.bind(
                jnp.where(iota == s_i, scv, 0), axes=(0,))
            sbase = s_i * (SCAP + BLK)
            nblocks = (subcnt + BLK - 1) // BLK

            def init_vec(r, _):
                for c in range(D // 16):
                    acc[r, pl.ds(c * 16, 16)] = neg_inf16
                return 0
            lax.fori_loop(0, SUBN + 1, init_vec, 0)

            def unpack(b, gi):
                def unpack_vec(k, _k):
                    p = lsub[pl.ds(sbase + b * BLK + k * 16, 16)]
                    gi[pl.ds(k * 16, 16)] = jnp.minimum(p & 0xFFFF, N - 1)
                    return 0
                lax.fori_loop(0, BLK // 16, unpack_vec, 0, unroll=True)

            def compute(b, gb):
                def do_group(g, _g):
                    p = lsub[pl.ds(sbase + b * BLK + g * 16, 16)]
                    ld = lax.shift_right_logical(p, 16)
                    rows = [lax.reduce_max_p.bind(
                        jnp.where(iota == l, ld, 0), axes=(0,))
                        for l in range(16)]
                    for l in range(16):
                        q = g * 16 + l
                        row = rows[l]
                        for c in range(D // 16):
                            gv = gb[q, pl.ds(c * 16, 16)]
                            av = acc[row, pl.ds(c * 16, 16)]
                            acc[row, pl.ds(c * 16, 16)] = jnp.maximum(av, gv)
                    return 0
                lax.fori_loop(0, BLK // 16, do_group, 0)

            def do_block(b, _b):
                even = (b % 2) == 0

                def slot(gi, gb, sem, ogi, ogb, osem):
                    pltpu.make_async_copy(b_hbm.at[gi], gb, sem).wait()

                    @pl.when(b + 1 < nblocks)
                    def _():
                        unpack(b + 1, ogi)
                        pltpu.make_async_copy(b_hbm.at[ogi], ogb,
                                              osem).start()
                    compute(b, gb)
                    return 0

                return lax.cond(even,
                                lambda: slot(gidx0, gbuf0, gsem0,
                                             gidx1, gbuf1, gsem1),
                                lambda: slot(gidx1, gbuf1, gsem1,
                                             gidx0, gbuf0, gsem0))

            @pl.when(nblocks > 0)
            def _():
                unpack(0, gidx0)
                pltpu.make_async_copy(b_hbm.at[gidx0], gbuf0, gsem0).start()

            lax.fori_loop(0, nblocks, do_block, 0)

            pltpu.sync_copy(acc.at[pl.ds(0, SUBN)],
                            m_hbm.at[pl.ds(lo + s_i * SUBN, SUBN)])
            return 0

        lax.fori_loop(0, NSUB, sub_range, 0)


@functools.partial(
    pl.kernel,
    compiler_params=pltpu.CompilerParams(needs_layout_passes=False),
    out_type=(jax.ShapeDtypeStruct((NPAD, D), jnp.float32),
              jax.ShapeDtypeStruct((NPAD, D), jnp.float32)),
    mesh=plsc.VectorSubcoreMesh(core_axis_name="c", subcore_axis_name="s",
                                num_cores=2, num_subcores=16),
    scratch_types=[
        pltpu.VMEM((EC,), jnp.int32),
        pltpu.VMEM((EC,), jnp.int32),
        pltpu.VMEM((EC,), jnp.int32),
        pltpu.VMEM((EC,), jnp.int32),
        pltpu.VMEM((16 * LSUB,), jnp.int32),
        pltpu.VMEM((NSUB * (SCAP + BLK),), jnp.int32),
        pltpu.VMEM((BLK,), jnp.int32),
        pltpu.VMEM((BLK,), jnp.int32),
        pltpu.VMEM((BLK, D), jnp.float32),
        pltpu.VMEM((BLK, D), jnp.float32),
        pltpu.VMEM((SUBN + 1, D), jnp.float32),
        pltpu.SemaphoreType.DMA,
        pltpu.SemaphoreType.DMA,
        pltpu.SemaphoreType.DMA,
        pltpu.SemaphoreType.DMA,
    ],
)
def _sc_segmax(*args):
    _sc_body(*args)


def kernel(x, edge_index_tp, edge_index_int, W1, b1, W2, b2):
    a1, a2, bm1, bm2 = _tc_matmuls(x, W1, b1, W2, b2)
    m1p, m2p = _sc_segmax(edge_index_tp[0], edge_index_tp[1],
                          edge_index_int[0], edge_index_int[1], bm1, bm2)
    return _tc_combine(a1, m1p[:N], a2, m2p[:N])


# speedup vs baseline: 4.1436x; 1.0002x over previous
import functools

import jax
import jax.numpy as jnp
from jax import lax
from jax.experimental import pallas as pl
from jax.experimental.pallas import tpu as pltpu
from jax.experimental.pallas import tpu_sc as plsc

N = 50000
D = 128
E = 400000

NW = 32
NPT = 1568
NPAD = NW * NPT
NSUB = 4
SUBN = NPT // NSUB
EC = 2000
NCHUNK = E // EC
BLK = 128
LSUB = 1088
SCAP = 3968

_NODE_BLK = 1000
_NEG_INF = float("-inf")


def _mm_body(x_ref, w1_ref, b1_ref, w2_ref, b2_ref,
             a1_ref, a2_ref, bo1_ref, bo2_ref):
    xb = x_ref[...]
    for w_ref, b_ref, a_out, b_out in (
            (w1_ref, b1_ref, a1_ref, bo1_ref),
            (w2_ref, b2_ref, a2_ref, bo2_ref)):
        w = w_ref[...]
        wa = w[:, :D]
        wb = w[:, D:]
        a_out[...] = jax.lax.dot_general(
            xb, wa - wb, (((1,), (1,)), ((), ())),
            preferred_element_type=jnp.float32,
            precision=jax.lax.Precision.HIGHEST) + b_ref[...]
        b_out[...] = jax.lax.dot_general(
            xb, wb, (((1,), (1,)), ((), ())),
            preferred_element_type=jnp.float32,
            precision=jax.lax.Precision.HIGHEST)


def _tc_matmuls(x, w1, b1, w2, b2):
    nblk = N // _NODE_BLK
    full = pl.BlockSpec((_NODE_BLK, D), lambda i: (i, 0))
    wspec = pl.BlockSpec((D, 2 * D), lambda i: (0, 0))
    bspec = pl.BlockSpec((D,), lambda i: (0,))
    return pl.pallas_call(
        _mm_body,
        grid=(nblk,),
        in_specs=[full, wspec, bspec, wspec, bspec],
        out_specs=[full, full, full, full],
        out_shape=[jax.ShapeDtypeStruct((N, D), jnp.float32)] * 4,
    )(x, w1, b1, w2, b2)


def _combine_body(a1_ref, m1_ref, a2_ref, m2_ref, o_ref):
    o_ref[...] = (jnp.maximum(a1_ref[...] + m1_ref[...], 0.0)
                  + jnp.maximum(a2_ref[...] + m2_ref[...], 0.0))


def _tc_combine(a1, m1, a2, m2):
    nblk = N // _NODE_BLK
    spec = pl.BlockSpec((_NODE_BLK, D), lambda i: (i, 0))
    return pl.pallas_call(
        _combine_body,
        grid=(nblk,),
        in_specs=[spec, spec, spec, spec],
        out_specs=spec,
        out_shape=jax.ShapeDtypeStruct((N, D), jnp.float32),
    )(a1, m1, a2, m2)


def _sc_body(src1_hbm, dst1_hbm, src2_hbm, dst2_hbm, b1_hbm, b2_hbm,
             m1_hbm, m2_hbm,
             dbuf0, dbuf1, sbuf0, sbuf1, lmain, lsub,
             gidx0, gidx1, gbuf0, gbuf1, acc, fsem0, fsem1, gsem0, gsem1):
    wid = lax.axis_index("s") * 2 + lax.axis_index("c")
    lo = wid * NPT
    iota = lax.iota(jnp.int32, 16)
    neg_inf16 = jnp.full((16,), _NEG_INF, jnp.float32)
    zeros16 = jnp.zeros((16,), jnp.int32)

    for tp in range(2):
        src_hbm = (src1_hbm, src2_hbm)[tp]
        dst_hbm = (dst1_hbm, dst2_hbm)[tp]
        b_hbm = (b1_hbm, b2_hbm)[tp]
        m_hbm = (m1_hbm, m2_hbm)[tp]

        def chunk_start(c, db, sb, sem):
            pltpu.make_async_copy(dst_hbm.at[pl.ds(c * EC, EC)], db,
                                  sem).start()
            pltpu.make_async_copy(src_hbm.at[pl.ds(c * EC, EC)], sb,
                                  sem).start()

        def chunk_wait(c, db, sb, sem):
            pltpu.make_async_copy(dst_hbm.at[pl.ds(c * EC, EC)], db,
                                  sem).wait()
            pltpu.make_async_copy(src_hbm.at[pl.ds(c * EC, EC)], sb,
                                  sem).wait()

        lanebase = iota * LSUB

        def scan_chunk(db, sb, cnt_v):
            def filt_vec(k, cnt):
                d = db[pl.ds(k * 16, 16)]
                s = sb[pl.ds(k * 16, 16)]
                ld = d - lo
                m = plsc.bitcast(ld, jnp.uint32) < jnp.uint32(NPT)
                m = m & (cnt < LSUB)
                plsc.store_scatter(lmain, [lanebase + cnt],
                                   s | (ld << 16), mask=m)
                return cnt + m.astype(jnp.int32)
            return lax.fori_loop(0, EC // 16, filt_vec, cnt_v, unroll=8)

        def filt_chunk(c, cnt_v):
            even = (c % 2) == 0

            def do_slot(db, sb, sem, odb, osb, osem):
                chunk_wait(c, db, sb, sem)

                @pl.when(c + 1 < NCHUNK)
                def _():
                    chunk_start(c + 1, odb, osb, osem)
                return scan_chunk(db, sb, cnt_v)

            r0 = lax.cond(even,
                          lambda: do_slot(dbuf0, sbuf0, fsem0,
                                          dbuf1, sbuf1, fsem1),
                          lambda: do_slot(dbuf1, sbuf1, fsem1,
                                          dbuf0, sbuf0, fsem0))
            return r0

        chunk_start(0, dbuf0, sbuf0, fsem0)
        cnt_v = lax.fori_loop(0, NCHUNK, filt_chunk, zeros16)

        def split_lane(l, cnts):
            bound = lax.reduce_max_p.bind(
                jnp.where(iota == l, cnt_v, 0), axes=(0,))

            def split_vec(k, cs):
                p = lmain[pl.ds(l * LSUB + k * 16, 16)]
                valid = iota < (bound - k * 16)
                ld = p >> 16
                new_cs = []
                for s_i in range(NSUB):
                    ms = valid & (ld >= s_i * SUBN) & (ld < (s_i + 1) * SUBN)
                    mi = ms.astype(jnp.int32)
                    pos = cs[s_i] + plsc.cumsum(mi) - mi
                    ms = ms & (pos < SCAP)
                    plsc.store_scatter(lsub, [pos + s_i * (SCAP + BLK)],
                                       p - ((s_i * SUBN) << 16), mask=ms)
                    new_cs.append(cs[s_i]
                                  + plsc.all_reduce_population_count(ms))
                return tuple(new_cs)

            return lax.fori_loop(0, (bound + 15) // 16, split_vec, cnts)

        cnts = lax.fori_loop(0, 16, split_lane, (zeros16,) * NSUB)

        padv = jnp.full((16,), SUBN << 16, jnp.int32)
        for s_i in range(NSUB):
            for k in range(BLK // 16):
                plsc.store_scatter(
                    lsub, [cnts[s_i] + s_i * (SCAP + BLK) + iota + k * 16],
                    padv)
        scv = zeros16
        for s_i in range(NSUB):
            scv = jnp.where(iota == s_i, cnts[s_i], scv)

        def sub_range(s_i, _s):
            subcnt = lax.reduce_max_p.bind(
                jnp.where(iota == s_i, scv, 0), axes=(0,))
            sbase = s_i * (SCAP + BLK)
            nblocks = (subcnt + BLK - 1) // BLK

            def init_vec(r, _):
                for c in range(D // 16):
                    acc[r, pl.ds(c * 16, 16)] = neg_inf16
                return 0
            lax.fori_loop(0, SUBN + 1, init_vec, 0)

            def unpack(b, gi):
                def unpack_vec(k, _k):
                    p = lsub[pl.ds(sbase + b * BLK + k * 16, 16)]
                    gi[pl.ds(k * 16, 16)] = jnp.minimum(p & 0xFFFF, N - 1)
                    return 0
                lax.fori_loop(0, BLK // 16, unpack_vec, 0, unroll=True)

            def compute(b, gb):
                def do_group(g, _g):
                    p = lsub[pl.ds(sbase + b * BLK + g * 16, 16)]
                    ld = lax.shift_right_logical(p, 16)
                    rows = [lax.reduce_max_p.bind(
                        jnp.where(iota == l, ld, 0), axes=(0,))
                        for l in range(16)]
                    for l in range(16):
                        q = g * 16 + l
                        row = rows[l]
                        for c in range(D // 16):
                            gv = gb[q, pl.ds(c * 16, 16)]
                            av = acc[row, pl.ds(c * 16, 16)]
                            acc[row, pl.ds(c * 16, 16)] = jnp.maximum(av, gv)
                    return 0
                lax.fori_loop(0, BLK // 16, do_group, 0)

            def do_block(b, _b):
                even = (b % 2) == 0

                def slot(gi, gb, sem, ogi, ogb, osem):
                    pltpu.make_async_copy(b_hbm.at[gi], gb, sem).wait()

                    @pl.when(b + 1 < nblocks)
                    def _():
                        unpack(b + 1, ogi)
                        pltpu.make_async_copy(b_hbm.at[ogi], ogb,
                                              osem).start()
                    compute(b, gb)
                    return 0

                return lax.cond(even,
                                lambda: slot(gidx0, gbuf0, gsem0,
                                             gidx1, gbuf1, gsem1),
                                lambda: slot(gidx1, gbuf1, gsem1,
                                             gidx0, gbuf0, gsem0))

            @pl.when(nblocks > 0)
            def _():
                unpack(0, gidx0)
                pltpu.make_async_copy(b_hbm.at[gidx0], gbuf0, gsem0).start()

            lax.fori_loop(0, nblocks, do_block, 0)

            pltpu.sync_copy(acc.at[pl.ds(0, SUBN)],
                            m_hbm.at[pl.ds(lo + s_i * SUBN, SUBN)])
            return 0

        lax.fori_loop(0, NSUB, sub_range, 0)


@functools.partial(
    pl.kernel,
    compiler_params=pltpu.CompilerParams(needs_layout_passes=False),
    out_type=(jax.ShapeDtypeStruct((NPAD, D), jnp.float32),
              jax.ShapeDtypeStruct((NPAD, D), jnp.float32)),
    mesh=plsc.VectorSubcoreMesh(core_axis_name="c", subcore_axis_name="s",
                                num_cores=2, num_subcores=16),
    scratch_types=[
        pltpu.VMEM((EC,), jnp.int32),
        pltpu.VMEM((EC,), jnp.int32),
        pltpu.VMEM((EC,), jnp.int32),
        pltpu.VMEM((EC,), jnp.int32),
        pltpu.VMEM((16 * LSUB,), jnp.int32),
        pltpu.VMEM((NSUB * (SCAP + BLK),), jnp.int32),
        pltpu.VMEM((BLK,), jnp.int32),
        pltpu.VMEM((BLK,), jnp.int32),
        pltpu.VMEM((BLK, D), jnp.float32),
        pltpu.VMEM((BLK, D), jnp.float32),
        pltpu.VMEM((SUBN + 1, D), jnp.float32),
        pltpu.SemaphoreType.DMA,
        pltpu.SemaphoreType.DMA,
        pltpu.SemaphoreType.DMA,
        pltpu.SemaphoreType.DMA,
    ],
)
def _sc_segmax(*args):
    _sc_body(*args)


def kernel(x, edge_index_tp, edge_index_int, W1, b1, W2, b2):
    a1, a2, bm1, bm2 = _tc_matmuls(x, W1, b1, W2, b2)
    m1p, m2p = _sc_segmax(edge_index_tp[0], edge_index_tp[1],
                          edge_index_int[0], edge_index_int[1], bm1, bm2)
    return _tc_combine(a1, m1p[:N], a2, m2p[:N])


# speedup vs baseline: 4.4003x; 1.0620x over previous
import functools

import jax
import jax.numpy as jnp
from jax import lax
from jax.experimental import pallas as pl
from jax.experimental.pallas import tpu as pltpu
from jax.experimental.pallas import tpu_sc as plsc

N = 50000
D = 128
E = 400000

NW = 32
NPT = 1568
NPAD = NW * NPT
NSUB = 4
SUBN = NPT // NSUB
EC = 2000
NCHUNK = E // EC
BLK = 128
LSUB = 1088
SCAP = 3968

_NODE_BLK = 1000
_NEG_INF = float("-inf")


def _mm_body(x_ref, w1_ref, b1_ref, w2_ref, b2_ref,
             a1_ref, a2_ref, bo1_ref, bo2_ref):
    xb = x_ref[...]
    for w_ref, b_ref, a_out, b_out in (
            (w1_ref, b1_ref, a1_ref, bo1_ref),
            (w2_ref, b2_ref, a2_ref, bo2_ref)):
        w = w_ref[...]
        wa = w[:, :D]
        wb = w[:, D:]
        a_out[...] = jax.lax.dot_general(
            xb, wa - wb, (((1,), (1,)), ((), ())),
            preferred_element_type=jnp.float32,
            precision=jax.lax.Precision.HIGHEST) + b_ref[...]
        b_out[...] = jax.lax.dot_general(
            xb, wb, (((1,), (1,)), ((), ())),
            preferred_element_type=jnp.float32,
            precision=jax.lax.Precision.HIGHEST)


def _tc_matmuls(x, w1, b1, w2, b2):
    nblk = N // _NODE_BLK
    full = pl.BlockSpec((_NODE_BLK, D), lambda i: (i, 0))
    wspec = pl.BlockSpec((D, 2 * D), lambda i: (0, 0))
    bspec = pl.BlockSpec((D,), lambda i: (0,))
    return pl.pallas_call(
        _mm_body,
        grid=(nblk,),
        in_specs=[full, wspec, bspec, wspec, bspec],
        out_specs=[full, full, full, full],
        out_shape=[jax.ShapeDtypeStruct((N, D), jnp.float32)] * 4,
    )(x, w1, b1, w2, b2)


def _combine_body(a1_ref, m1_ref, a2_ref, m2_ref, o_ref):
    o_ref[...] = (jnp.maximum(a1_ref[...] + m1_ref[...], 0.0)
                  + jnp.maximum(a2_ref[...] + m2_ref[...], 0.0))


def _tc_combine(a1, m1, a2, m2):
    nblk = N // _NODE_BLK
    spec = pl.BlockSpec((_NODE_BLK, D), lambda i: (i, 0))
    return pl.pallas_call(
        _combine_body,
        grid=(nblk,),
        in_specs=[spec, spec, spec, spec],
        out_specs=spec,
        out_shape=jax.ShapeDtypeStruct((N, D), jnp.float32),
    )(a1, m1, a2, m2)


def _sc_body(src1_hbm, dst1_hbm, src2_hbm, dst2_hbm, b1_hbm, b2_hbm,
             m1_hbm, m2_hbm,
             dbuf0, dbuf1, sbuf0, sbuf1, lmain, lsub,
             gidx0, gidx1, gbuf0, gbuf1, acc, fsem0, fsem1, gsem0, gsem1):
    wid = lax.axis_index("s") * 2 + lax.axis_index("c")
    lo = wid * NPT
    iota = lax.iota(jnp.int32, 16)
    neg_inf16 = jnp.full((16,), _NEG_INF, jnp.float32)
    zeros16 = jnp.zeros((16,), jnp.int32)

    for tp in range(2):
        src_hbm = (src1_hbm, src2_hbm)[tp]
        dst_hbm = (dst1_hbm, dst2_hbm)[tp]
        b_hbm = (b1_hbm, b2_hbm)[tp]
        m_hbm = (m1_hbm, m2_hbm)[tp]

        def chunk_start(c, db, sb, sem):
            pltpu.make_async_copy(dst_hbm.at[pl.ds(c * EC, EC)], db,
                                  sem).start()
            pltpu.make_async_copy(src_hbm.at[pl.ds(c * EC, EC)], sb,
                                  sem).start()

        def chunk_wait(c, db, sb, sem):
            pltpu.make_async_copy(dst_hbm.at[pl.ds(c * EC, EC)], db,
                                  sem).wait()
            pltpu.make_async_copy(src_hbm.at[pl.ds(c * EC, EC)], sb,
                                  sem).wait()

        lanebase = iota * LSUB

        def scan_chunk(db, sb, cnt_v):
            def filt_vec(k, cnt):
                d = db[pl.ds(k * 16, 16)]
                s = sb[pl.ds(k * 16, 16)]
                ld = d - lo
                m = plsc.bitcast(ld, jnp.uint32) < jnp.uint32(NPT)
                m = m & (cnt < LSUB)
                plsc.store_scatter(lmain, [lanebase + cnt],
                                   s | (ld << 16), mask=m)
                return cnt + m.astype(jnp.int32)
            return plsc.parallel_loop(0, EC // 16, unroll=4,
                                      carry=cnt_v)(filt_vec)

        def filt_chunk(c, cnt_v):
            even = (c % 2) == 0

            def do_slot(db, sb, sem, odb, osb, osem):
                chunk_wait(c, db, sb, sem)

                @pl.when(c + 1 < NCHUNK)
                def _():
                    chunk_start(c + 1, odb, osb, osem)
                return scan_chunk(db, sb, cnt_v)

            r0 = lax.cond(even,
                          lambda: do_slot(dbuf0, sbuf0, fsem0,
                                          dbuf1, sbuf1, fsem1),
                          lambda: do_slot(dbuf1, sbuf1, fsem1,
                                          dbuf0, sbuf0, fsem0))
            return r0

        chunk_start(0, dbuf0, sbuf0, fsem0)
        cnt_v = lax.fori_loop(0, NCHUNK, filt_chunk, zeros16)

        def split_lane(l, cnts):
            bound = lax.reduce_max_p.bind(
                jnp.where(iota == l, cnt_v, 0), axes=(0,))

            def split_vec(k, cs):
                p = lmain[pl.ds(l * LSUB + k * 16, 16)]
                valid = iota < (bound - k * 16)
                ld = p >> 16
                new_cs = []
                for s_i in range(NSUB):
                    ms = valid & (ld >= s_i * SUBN) & (ld < (s_i + 1) * SUBN)
                    mi = ms.astype(jnp.int32)
                    pos = cs[s_i] + plsc.cumsum(mi) - mi
                    ms = ms & (pos < SCAP)
                    plsc.store_scatter(lsub, [pos + s_i * (SCAP + BLK)],
                                       p - ((s_i * SUBN) << 16), mask=ms)
                    new_cs.append(cs[s_i]
                                  + plsc.all_reduce_population_count(ms))
                return tuple(new_cs)

            return lax.fori_loop(0, (bound + 15) // 16, split_vec, cnts)

        cnts = lax.fori_loop(0, 16, split_lane, (zeros16,) * NSUB)

        padv = jnp.full((16,), SUBN << 16, jnp.int32)
        for s_i in range(NSUB):
            for k in range(BLK // 16):
                plsc.store_scatter(
                    lsub, [cnts[s_i] + s_i * (SCAP + BLK) + iota + k * 16],
                    padv)
        scv = zeros16
        for s_i in range(NSUB):
            scv = jnp.where(iota == s_i, cnts[s_i], scv)

        def sub_range(s_i, _s):
            subcnt = lax.reduce_max_p.bind(
                jnp.where(iota == s_i, scv, 0), axes=(0,))
            sbase = s_i * (SCAP + BLK)
            nblocks = (subcnt + BLK - 1) // BLK

            def init_vec(r):
                for c in range(D // 16):
                    acc[r, pl.ds(c * 16, 16)] = neg_inf16
            plsc.parallel_loop(0, SUBN + 1)(init_vec)

            def unpack(b, gi):
                def unpack_vec(k):
                    p = lsub[pl.ds(sbase + b * BLK + k * 16, 16)]
                    gi[pl.ds(k * 16, 16)] = jnp.minimum(p & 0xFFFF, N - 1)
                plsc.parallel_loop(0, BLK // 16, unroll=2)(unpack_vec)

            def compute(b, gb):
                def do_group(g, _g):
                    p = lsub[pl.ds(sbase + b * BLK + g * 16, 16)]
                    ld = lax.shift_right_logical(p, 16)
                    rows = [lax.reduce_max_p.bind(
                        jnp.where(iota == l, ld, 0), axes=(0,))
                        for l in range(16)]
                    for l in range(16):
                        q = g * 16 + l
                        row = rows[l]
                        for c in range(D // 16):
                            gv = gb[q, pl.ds(c * 16, 16)]
                            av = acc[row, pl.ds(c * 16, 16)]
                            acc[row, pl.ds(c * 16, 16)] = jnp.maximum(av, gv)
                    return 0
                lax.fori_loop(0, BLK // 16, do_group, 0)

            def do_block(b, _b):
                even = (b % 2) == 0

                def slot(gi, gb, sem, ogi, ogb, osem):
                    pltpu.make_async_copy(b_hbm.at[gi], gb, sem).wait()

                    @pl.when(b + 1 < nblocks)
                    def _():
                        unpack(b + 1, ogi)
                        pltpu.make_async_copy(b_hbm.at[ogi], ogb,
                                              osem).start()
                    compute(b, gb)
                    return 0

                return lax.cond(even,
                                lambda: slot(gidx0, gbuf0, gsem0,
                                             gidx1, gbuf1, gsem1),
                                lambda: slot(gidx1, gbuf1, gsem1,
                                             gidx0, gbuf0, gsem0))

            @pl.when(nblocks > 0)
            def _():
                unpack(0, gidx0)
                pltpu.make_async_copy(b_hbm.at[gidx0], gbuf0, gsem0).start()

            lax.fori_loop(0, nblocks, do_block, 0)

            pltpu.sync_copy(acc.at[pl.ds(0, SUBN)],
                            m_hbm.at[pl.ds(lo + s_i * SUBN, SUBN)])
            return 0

        lax.fori_loop(0, NSUB, sub_range, 0)


@functools.partial(
    pl.kernel,
    compiler_params=pltpu.CompilerParams(needs_layout_passes=False),
    out_type=(jax.ShapeDtypeStruct((NPAD, D), jnp.float32),
              jax.ShapeDtypeStruct((NPAD, D), jnp.float32)),
    mesh=plsc.VectorSubcoreMesh(core_axis_name="c", subcore_axis_name="s",
                                num_cores=2, num_subcores=16),
    scratch_types=[
        pltpu.VMEM((EC,), jnp.int32),
        pltpu.VMEM((EC,), jnp.int32),
        pltpu.VMEM((EC,), jnp.int32),
        pltpu.VMEM((EC,), jnp.int32),
        pltpu.VMEM((16 * LSUB,), jnp.int32),
        pltpu.VMEM((NSUB * (SCAP + BLK),), jnp.int32),
        pltpu.VMEM((BLK,), jnp.int32),
        pltpu.VMEM((BLK,), jnp.int32),
        pltpu.VMEM((BLK, D), jnp.float32),
        pltpu.VMEM((BLK, D), jnp.float32),
        pltpu.VMEM((SUBN + 1, D), jnp.float32),
        pltpu.SemaphoreType.DMA,
        pltpu.SemaphoreType.DMA,
        pltpu.SemaphoreType.DMA,
        pltpu.SemaphoreType.DMA,
    ],
)
def _sc_segmax(*args):
    _sc_body(*args)


def kernel(x, edge_index_tp, edge_index_int, W1, b1, W2, b2):
    a1, a2, bm1, bm2 = _tc_matmuls(x, W1, b1, W2, b2)
    m1p, m2p = _sc_segmax(edge_index_tp[0], edge_index_tp[1],
                          edge_index_int[0], edge_index_int[1], bm1, bm2)
    return _tc_combine(a1, m1p[:N], a2, m2p[:N])


# speedup vs baseline: 4.4116x; 1.0026x over previous
import functools

import jax
import jax.numpy as jnp
from jax import lax
from jax.experimental import pallas as pl
from jax.experimental.pallas import tpu as pltpu
from jax.experimental.pallas import tpu_sc as plsc

N = 50000
D = 128
E = 400000

NW = 32
NPT = 1568
NPAD = NW * NPT
NSUB = 4
SUBN = NPT // NSUB
EC = 2000
NCHUNK = E // EC
BLK = 128
LSUB = 1088
SCAP = 3968

_NODE_BLK = 1000
_NEG_INF = float("-inf")


def _mm_body(x_ref, w1_ref, b1_ref, w2_ref, b2_ref,
             a1_ref, a2_ref, bo1_ref, bo2_ref):
    xb = x_ref[...]
    for w_ref, b_ref, a_out, b_out in (
            (w1_ref, b1_ref, a1_ref, bo1_ref),
            (w2_ref, b2_ref, a2_ref, bo2_ref)):
        w = w_ref[...]
        wa = w[:, :D]
        wb = w[:, D:]
        a_out[...] = jax.lax.dot_general(
            xb, wa - wb, (((1,), (1,)), ((), ())),
            preferred_element_type=jnp.float32,
            precision=jax.lax.Precision.HIGHEST) + b_ref[...]
        b_out[...] = jax.lax.dot_general(
            xb, wb, (((1,), (1,)), ((), ())),
            preferred_element_type=jnp.float32,
            precision=jax.lax.Precision.HIGHEST)


def _tc_matmuls(x, w1, b1, w2, b2):
    nblk = N // _NODE_BLK
    full = pl.BlockSpec((_NODE_BLK, D), lambda i: (i, 0))
    wspec = pl.BlockSpec((D, 2 * D), lambda i: (0, 0))
    bspec = pl.BlockSpec((D,), lambda i: (0,))
    return pl.pallas_call(
        _mm_body,
        grid=(nblk,),
        in_specs=[full, wspec, bspec, wspec, bspec],
        out_specs=[full, full, full, full],
        out_shape=[jax.ShapeDtypeStruct((N, D), jnp.float32)] * 4,
    )(x, w1, b1, w2, b2)


def _combine_body(a1_ref, m1_ref, a2_ref, m2_ref, o_ref):
    o_ref[...] = (jnp.maximum(a1_ref[...] + m1_ref[...], 0.0)
                  + jnp.maximum(a2_ref[...] + m2_ref[...], 0.0))


def _tc_combine(a1, m1, a2, m2):
    nblk = N // _NODE_BLK
    spec = pl.BlockSpec((_NODE_BLK, D), lambda i: (i, 0))
    return pl.pallas_call(
        _combine_body,
        grid=(nblk,),
        in_specs=[spec, spec, spec, spec],
        out_specs=spec,
        out_shape=jax.ShapeDtypeStruct((N, D), jnp.float32),
    )(a1, m1, a2, m2)


def _sc_body(src1_hbm, dst1_hbm, src2_hbm, dst2_hbm, b1_hbm, b2_hbm,
             m1_hbm, m2_hbm,
             dbuf0, dbuf1, sbuf0, sbuf1, lmain, lsub,
             gidx0, gidx1, gbuf0, gbuf1, acc, fsem0, fsem1, gsem0, gsem1):
    wid = lax.axis_index("s") * 2 + lax.axis_index("c")
    lo = wid * NPT
    iota = lax.iota(jnp.int32, 16)
    neg_inf16 = jnp.full((16,), _NEG_INF, jnp.float32)
    zeros16 = jnp.zeros((16,), jnp.int32)

    for tp in range(2):
        src_hbm = (src1_hbm, src2_hbm)[tp]
        dst_hbm = (dst1_hbm, dst2_hbm)[tp]
        b_hbm = (b1_hbm, b2_hbm)[tp]
        m_hbm = (m1_hbm, m2_hbm)[tp]

        def chunk_start(c, db, sb, sem):
            pltpu.make_async_copy(dst_hbm.at[pl.ds(c * EC, EC)], db,
                                  sem).start()
            pltpu.make_async_copy(src_hbm.at[pl.ds(c * EC, EC)], sb,
                                  sem).start()

        def chunk_wait(c, db, sb, sem):
            pltpu.make_async_copy(dst_hbm.at[pl.ds(c * EC, EC)], db,
                                  sem).wait()
            pltpu.make_async_copy(src_hbm.at[pl.ds(c * EC, EC)], sb,
                                  sem).wait()

        lanebase = iota * LSUB

        def scan_chunk(db, sb, cnt_v):
            def filt_vec(k, cnt):
                d = db[pl.ds(k * 16, 16)]
                s = sb[pl.ds(k * 16, 16)]
                ld = d - lo
                m = plsc.bitcast(ld, jnp.uint32) < jnp.uint32(NPT)
                m = m & (cnt < LSUB)
                plsc.store_scatter(lmain, [lanebase + cnt],
                                   s | (ld << 16), mask=m)
                return cnt + m.astype(jnp.int32)
            return plsc.parallel_loop(0, EC // 16, unroll=4,
                                      carry=cnt_v)(filt_vec)

        def filt_chunk(c, cnt_v):
            even = (c % 2) == 0

            def do_slot(db, sb, sem, odb, osb, osem):
                chunk_wait(c, db, sb, sem)

                @pl.when(c + 1 < NCHUNK)
                def _():
                    chunk_start(c + 1, odb, osb, osem)
                return scan_chunk(db, sb, cnt_v)

            r0 = lax.cond(even,
                          lambda: do_slot(dbuf0, sbuf0, fsem0,
                                          dbuf1, sbuf1, fsem1),
                          lambda: do_slot(dbuf1, sbuf1, fsem1,
                                          dbuf0, sbuf0, fsem0))
            return r0

        chunk_start(0, dbuf0, sbuf0, fsem0)
        cnt_v = lax.fori_loop(0, NCHUNK, filt_chunk, zeros16)

        def split_lane(l, cnts):
            bound = lax.reduce_max_p.bind(
                jnp.where(iota == l, cnt_v, 0), axes=(0,))

            def split_vec(k, cs):
                p = lmain[pl.ds(l * LSUB + k * 16, 16)]
                valid = iota < (bound - k * 16)
                ld = p >> 16
                new_cs = []
                for s_i in range(NSUB):
                    ms = valid & (ld >= s_i * SUBN) & (ld < (s_i + 1) * SUBN)
                    mi = ms.astype(jnp.int32)
                    pos = cs[s_i] + plsc.cumsum(mi) - mi
                    ms = ms & (pos < SCAP)
                    plsc.store_scatter(lsub, [pos + s_i * (SCAP + BLK)],
                                       p - ((s_i * SUBN) << 16), mask=ms)
                    new_cs.append(cs[s_i]
                                  + plsc.all_reduce_population_count(ms))
                return tuple(new_cs)

            return lax.fori_loop(0, (bound + 15) // 16, split_vec, cnts)

        cnts = lax.fori_loop(0, 16, split_lane, (zeros16,) * NSUB)

        padv = jnp.full((16,), SUBN << 16, jnp.int32)
        for s_i in range(NSUB):
            for k in range(BLK // 16):
                plsc.store_scatter(
                    lsub, [cnts[s_i] + s_i * (SCAP + BLK) + iota + k * 16],
                    padv)
        scv = zeros16
        for s_i in range(NSUB):
            scv = jnp.where(iota == s_i, cnts[s_i], scv)

        def sub_range(s_i, _s):
            subcnt = lax.reduce_max_p.bind(
                jnp.where(iota == s_i, scv, 0), axes=(0,))
            sbase = s_i * (SCAP + BLK)
            nblocks = (subcnt + BLK - 1) // BLK

            def init_vec(r):
                for c in range(D // 16):
                    acc[r, pl.ds(c * 16, 16)] = neg_inf16
            plsc.parallel_loop(0, SUBN + 1)(init_vec)

            def unpack(b, gi):
                def unpack_vec(k):
                    p = lsub[pl.ds(sbase + b * BLK + k * 16, 16)]
                    gi[pl.ds(k * 16, 16)] = jnp.minimum(p & 0xFFFF, N - 1)
                plsc.parallel_loop(0, BLK // 16, unroll=2)(unpack_vec)

            def compute(b, gb):
                def do_group(g, _g):
                    p = lsub[pl.ds(sbase + b * BLK + g * 16, 16)]
                    ld = lax.shift_right_logical(p, 16)
                    rows = [lax.reduce_max_p.bind(
                        jnp.where(iota == l, ld, 0), axes=(0,))
                        for l in range(16)]
                    for l in range(16):
                        q = g * 16 + l
                        row = rows[l]
                        for c in range(D // 16):
                            gv = gb[q, pl.ds(c * 16, 16)]
                            av = acc[row, pl.ds(c * 16, 16)]
                            acc[row, pl.ds(c * 16, 16)] = jnp.maximum(av, gv)
                    return 0
                lax.fori_loop(0, BLK // 16, do_group, 0)

            def do_block(b, _b):
                even = (b % 2) == 0

                def slot(gi, gb, sem, ogi, ogb, osem):
                    pltpu.make_async_copy(b_hbm.at[gi], gb, sem).wait()

                    @pl.when(b + 1 < nblocks)
                    def _():
                        unpack(b + 1, ogi)
                        pltpu.make_async_copy(b_hbm.at[ogi], ogb,
                                              osem).start()
                    compute(b, gb)
                    return 0

                return lax.cond(even,
                                lambda: slot(gidx0, gbuf0, gsem0,
                                             gidx1, gbuf1, gsem1),
                                lambda: slot(gidx1, gbuf1, gsem1,
                                             gidx0, gbuf0, gsem0))

            @pl.when(nblocks > 0)
            def _():
                unpack(0, gidx0)
                pltpu.make_async_copy(b_hbm.at[gidx0], gbuf0, gsem0).start()

            lax.fori_loop(0, nblocks, do_block, 0)

            pltpu.sync_copy(acc.at[pl.ds(0, SUBN)],
                            m_hbm.at[pl.ds(lo + s_i * SUBN, SUBN)])
            return 0

        lax.fori_loop(0, NSUB, sub_range, 0)


@functools.partial(
    pl.kernel,
    compiler_params=pltpu.CompilerParams(needs_layout_passes=False,
                                         disable_bounds_checks=True),
    out_type=(jax.ShapeDtypeStruct((NPAD, D), jnp.float32),
              jax.ShapeDtypeStruct((NPAD, D), jnp.float32)),
    mesh=plsc.VectorSubcoreMesh(core_axis_name="c", subcore_axis_name="s",
                                num_cores=2, num_subcores=16),
    scratch_types=[
        pltpu.VMEM((EC,), jnp.int32),
        pltpu.VMEM((EC,), jnp.int32),
        pltpu.VMEM((EC,), jnp.int32),
        pltpu.VMEM((EC,), jnp.int32),
        pltpu.VMEM((16 * LSUB,), jnp.int32),
        pltpu.VMEM((NSUB * (SCAP + BLK),), jnp.int32),
        pltpu.VMEM((BLK,), jnp.int32),
        pltpu.VMEM((BLK,), jnp.int32),
        pltpu.VMEM((BLK, D), jnp.float32),
        pltpu.VMEM((BLK, D), jnp.float32),
        pltpu.VMEM((SUBN + 1, D), jnp.float32),
        pltpu.SemaphoreType.DMA,
        pltpu.SemaphoreType.DMA,
        pltpu.SemaphoreType.DMA,
        pltpu.SemaphoreType.DMA,
    ],
)
def _sc_segmax(*args):
    _sc_body(*args)


def kernel(x, edge_index_tp, edge_index_int, W1, b1, W2, b2):
    a1, a2, bm1, bm2 = _tc_matmuls(x, W1, b1, W2, b2)
    m1p, m2p = _sc_segmax(edge_index_tp[0], edge_index_tp[1],
                          edge_index_int[0], edge_index_int[1], bm1, bm2)
    return _tc_combine(a1, m1p[:N], a2, m2p[:N])
